# baseline (speedup 1.0000x reference)
import jax
import jax.numpy as jnp
from jax.experimental import pallas as pl
from jax.experimental.pallas import tpu as pltpu

_N_CHUNKS = 8


def _copy_body(emb, lab, eq, lq, out_eq, out_lq, sem_q, sem_e, sem_l, sem_lb):
    B, S = emb.shape[0], eq.shape[0]
    tail = S - B
    chunk = tail // _N_CHUNKS
    qcopies = [
        pltpu.make_async_copy(
            eq.at[pl.ds(B + i * chunk, chunk)],
            out_eq.at[pl.ds(B + i * chunk, chunk)],
            sem_q.at[i],
        )
        for i in range(_N_CHUNKS)
    ]
    ecopy = pltpu.make_async_copy(emb, out_eq.at[pl.ds(0, B)], sem_e)
    rl = lab.shape[0]
    ltail = lq.shape[0] - rl
    lcopy = pltpu.make_async_copy(
        lq.at[pl.ds(rl, ltail)], out_lq.at[pl.ds(rl, ltail)], sem_l
    )
    lbcopy = pltpu.make_async_copy(lab, out_lq.at[pl.ds(0, rl)], sem_lb)
    for c in qcopies:
        c.start()
    ecopy.start()
    lcopy.start()
    lbcopy.start()
    for c in qcopies:
        c.wait()
    ecopy.wait()
    lcopy.wait()
    lbcopy.wait()


def kernel(embeddings, labels, embed_queue, label_queue):
    B, D = embeddings.shape
    S = embed_queue.shape[0]
    lab2 = labels.reshape(B // 128, 128)
    lq2 = label_queue.reshape(S // 128, 128)
    out_eq, out_lq = pl.pallas_call(
        _copy_body,
        in_specs=[pl.BlockSpec(memory_space=pl.ANY)] * 4,
        out_specs=[pl.BlockSpec(memory_space=pl.ANY)] * 2,
        out_shape=[
            jax.ShapeDtypeStruct(embed_queue.shape, embed_queue.dtype),
            jax.ShapeDtypeStruct(lq2.shape, lq2.dtype),
        ],
        scratch_shapes=[
            pltpu.SemaphoreType.DMA((_N_CHUNKS,)),
            pltpu.SemaphoreType.DMA,
            pltpu.SemaphoreType.DMA,
            pltpu.SemaphoreType.DMA,
        ],
    )(embeddings, lab2, embed_queue, lq2)
    new_ptr = jnp.array([B % S], dtype=jnp.int32)
    return out_eq, out_lq.reshape(S), new_ptr


# speedup vs baseline: 17.5437x; 17.5437x over previous
import jax
import jax.numpy as jnp
from jax.experimental import pallas as pl
from jax.experimental.pallas import tpu as pltpu

_BLK = 2048


def _copy_body(emb, lab, eq, lq, out_eq, out_lq, sem_e, sem_l):
    i = pl.program_id(0)
    r = out_eq.shape[0]
    rl = out_lq.shape[0]
    nb_emb = emb.shape[0] // r

    @pl.when(i < nb_emb)
    def _():
        ce = pltpu.make_async_copy(emb.at[pl.ds(i * r, r)], out_eq, sem_e)
        cl = pltpu.make_async_copy(lab.at[pl.ds(i * rl, rl)], out_lq, sem_l)
        ce.start()
        cl.start()
        ce.wait()
        cl.wait()

    @pl.when(i >= nb_emb)
    def _():
        ce = pltpu.make_async_copy(eq.at[pl.ds(i * r, r)], out_eq, sem_e)
        cl = pltpu.make_async_copy(lq.at[pl.ds(i * rl, rl)], out_lq, sem_l)
        ce.start()
        cl.start()
        ce.wait()
        cl.wait()


def kernel(embeddings, labels, embed_queue, label_queue):
    B, D = embeddings.shape
    S = embed_queue.shape[0]
    grid = S // _BLK
    blk_l = (S // 128) // grid
    lab2 = labels.reshape(B // 128, 128)
    lq2 = label_queue.reshape(S // 128, 128)
    out_eq, out_lq = pl.pallas_call(
        _copy_body,
        grid=(grid,),
        in_specs=[pl.BlockSpec(memory_space=pl.ANY)] * 4,
        out_specs=[
            pl.BlockSpec((_BLK, D), lambda i: (i, 0)),
            pl.BlockSpec((blk_l, 128), lambda i: (i, 0)),
        ],
        out_shape=[
            jax.ShapeDtypeStruct(embed_queue.shape, embed_queue.dtype),
            jax.ShapeDtypeStruct(lq2.shape, lq2.dtype),
        ],
        scratch_shapes=[
            pltpu.SemaphoreType.DMA,
            pltpu.SemaphoreType.DMA,
        ],
    )(embeddings, lab2, embed_queue, lq2)
    new_ptr = jnp.array([B % S], dtype=jnp.int32)
    return out_eq, out_lq.reshape(S), new_ptr


# speedup vs baseline: 19.2629x; 1.0980x over previous
import jax
import jax.numpy as jnp
from jax.experimental import pallas as pl
from jax.experimental.pallas import tpu as pltpu

_BLK = 1024


def _copy_body(emb, lab, eq, lq, out_eq, out_lq, sem_e, sem_l):
    i = pl.program_id(0)
    r = out_eq.shape[0]
    rl = out_lq.shape[0]
    nb_emb = emb.shape[0] // r

    @pl.when(i < nb_emb)
    def _():
        ce = pltpu.make_async_copy(emb.at[pl.ds(i * r, r)], out_eq, sem_e)
        cl = pltpu.make_async_copy(lab.at[pl.ds(i * rl, rl)], out_lq, sem_l)
        ce.start()
        cl.start()
        ce.wait()
        cl.wait()

    @pl.when(i >= nb_emb)
    def _():
        out_eq[...] = eq[...]
        out_lq[...] = lq[...]


def kernel(embeddings, labels, embed_queue, label_queue):
    B, D = embeddings.shape
    S = embed_queue.shape[0]
    grid = S // _BLK
    blk_l = (S // 128) // grid
    lab2 = labels.reshape(B // 128, 128)
    lq2 = label_queue.reshape(S // 128, 128)
    out_eq, out_lq = pl.pallas_call(
        _copy_body,
        grid=(grid,),
        in_specs=[
            pl.BlockSpec(memory_space=pl.ANY),
            pl.BlockSpec(memory_space=pl.ANY),
            pl.BlockSpec((_BLK, D), lambda i: (i, 0)),
            pl.BlockSpec((blk_l, 128), lambda i: (i, 0)),
        ],
        out_specs=[
            pl.BlockSpec((_BLK, D), lambda i: (i, 0)),
            pl.BlockSpec((blk_l, 128), lambda i: (i, 0)),
        ],
        out_shape=[
            jax.ShapeDtypeStruct(embed_queue.shape, embed_queue.dtype),
            jax.ShapeDtypeStruct(lq2.shape, lq2.dtype),
        ],
        scratch_shapes=[
            pltpu.SemaphoreType.DMA,
            pltpu.SemaphoreType.DMA,
        ],
    )(embeddings, lab2, embed_queue, lq2)
    new_ptr = jnp.array([B % S], dtype=jnp.int32)
    return out_eq, out_lq.reshape(S), new_ptr


# speedup vs baseline: 34.2777x; 1.7795x over previous
import jax
import jax.numpy as jnp
from jax.experimental import pallas as pl
from jax.experimental.pallas import tpu as pltpu

_R = 1024
_NBUF = 8
_K = 4


def _copy_body(emb, lab, eq, lq, out_eq, out_lq, vb, vlab, fsem, dsem, lsem):
    S, D = out_eq.shape
    B = emb.shape[0]
    nb = S // _R
    nb_emb = B // _R

    fills = [
        pltpu.make_async_copy(
            (emb if b < nb_emb else eq).at[pl.ds(b * _R, _R)],
            vb.at[b % _NBUF],
            fsem.at[b % _NBUF],
        )
        for b in range(nb)
    ]
    drains = [
        pltpu.make_async_copy(
            vb.at[b % _NBUF],
            out_eq.at[pl.ds(b * _R, _R)],
            dsem.at[b % _NBUF],
        )
        for b in range(nb)
    ]
    rl = lab.shape[0]
    ltail = lq.shape[0] - rl
    lfill1 = pltpu.make_async_copy(lab, vlab.at[pl.ds(0, rl)], lsem.at[0])
    lfill2 = pltpu.make_async_copy(
        lq.at[pl.ds(rl, ltail)], vlab.at[pl.ds(rl, ltail)], lsem.at[0]
    )
    ldrain = pltpu.make_async_copy(vlab, out_lq, lsem.at[1])

    lfill1.start()
    lfill2.start()
    for b in range(_NBUF):
        fills[b].start()
    lfill1.wait()
    lfill2.wait()
    ldrain.start()
    for b in range(nb):
        fills[b].wait()
        drains[b].start()
        j = b - _K
        if j >= 0 and j + _NBUF < nb:
            drains[j].wait()
            fills[j + _NBUF].start()
    waited = [j for j in range(nb) if j + _NBUF < nb and j <= nb - 1 - _K]
    first_unwaited = (waited[-1] + 1) if waited else 0
    for b in range(first_unwaited, nb):
        drains[b].wait()
    ldrain.wait()


def kernel(embeddings, labels, embed_queue, label_queue):
    B, D = embeddings.shape
    S = embed_queue.shape[0]
    lab2 = labels.reshape(B // 128, 128)
    lq2 = label_queue.reshape(S // 128, 128)
    out_eq, out_lq = pl.pallas_call(
        _copy_body,
        in_specs=[pl.BlockSpec(memory_space=pl.ANY)] * 4,
        out_specs=[pl.BlockSpec(memory_space=pl.ANY)] * 2,
        out_shape=[
            jax.ShapeDtypeStruct(embed_queue.shape, embed_queue.dtype),
            jax.ShapeDtypeStruct(lq2.shape, lq2.dtype),
        ],
        scratch_shapes=[
            pltpu.VMEM((_NBUF, _R, D), embed_queue.dtype),
            pltpu.VMEM((S // 128, 128), label_queue.dtype),
            pltpu.SemaphoreType.DMA((_NBUF,)),
            pltpu.SemaphoreType.DMA((_NBUF,)),
            pltpu.SemaphoreType.DMA((2,)),
        ],
    )(embeddings, lab2, embed_queue, lq2)
    new_ptr = jnp.array([B % S], dtype=jnp.int32)
    return out_eq, out_lq.reshape(S), new_ptr


# speedup vs baseline: 35.4595x; 1.0345x over previous
import jax
import jax.numpy as jnp
from jax.experimental import pallas as pl
from jax.experimental.pallas import tpu as pltpu

_R = 512
_NBUF = 16
_K = 8


def _copy_body(emb, lab, eq, lq, out_eq, out_lq, vb, vlab, fsem, dsem, lsem):
    S, D = out_eq.shape
    B = emb.shape[0]
    nb = S // _R
    nb_emb = B // _R

    fills = [
        pltpu.make_async_copy(
            (emb if b < nb_emb else eq).at[pl.ds(b * _R, _R)],
            vb.at[b % _NBUF],
            fsem.at[b % _NBUF],
        )
        for b in range(nb)
    ]
    drains = [
        pltpu.make_async_copy(
            vb.at[b % _NBUF],
            out_eq.at[pl.ds(b * _R, _R)],
            dsem.at[b % _NBUF],
        )
        for b in range(nb)
    ]
    rl = lab.shape[0]
    ltail = lq.shape[0] - rl
    lfill1 = pltpu.make_async_copy(lab, vlab.at[pl.ds(0, rl)], lsem.at[0])
    lfill2 = pltpu.make_async_copy(
        lq.at[pl.ds(rl, ltail)], vlab.at[pl.ds(rl, ltail)], lsem.at[0]
    )
    ldrain = pltpu.make_async_copy(vlab, out_lq, lsem.at[1])

    lfill1.start()
    lfill2.start()
    for b in range(_NBUF):
        fills[b].start()
    lfill1.wait()
    lfill2.wait()
    ldrain.start()
    for b in range(nb):
        fills[b].wait()
        drains[b].start()
        j = b - _K
        if j >= 0 and j + _NBUF < nb:
            drains[j].wait()
            fills[j + _NBUF].start()
    waited = [j for j in range(nb) if j + _NBUF < nb and j <= nb - 1 - _K]
    first_unwaited = (waited[-1] + 1) if waited else 0
    for b in range(first_unwaited, nb):
        drains[b].wait()
    ldrain.wait()


def kernel(embeddings, labels, embed_queue, label_queue):
    B, D = embeddings.shape
    S = embed_queue.shape[0]
    lab2 = labels.reshape(B // 128, 128)
    lq2 = label_queue.reshape(S // 128, 128)
    out_eq, out_lq = pl.pallas_call(
        _copy_body,
        in_specs=[pl.BlockSpec(memory_space=pl.ANY)] * 4,
        out_specs=[pl.BlockSpec(memory_space=pl.ANY)] * 2,
        out_shape=[
            jax.ShapeDtypeStruct(embed_queue.shape, embed_queue.dtype),
            jax.ShapeDtypeStruct(lq2.shape, lq2.dtype),
        ],
        scratch_shapes=[
            pltpu.VMEM((_NBUF, _R, D), embed_queue.dtype),
            pltpu.VMEM((S // 128, 128), label_queue.dtype),
            pltpu.SemaphoreType.DMA((_NBUF,)),
            pltpu.SemaphoreType.DMA((_NBUF,)),
            pltpu.SemaphoreType.DMA((2,)),
        ],
    )(embeddings, lab2, embed_queue, lq2)
    new_ptr = jnp.array([B % S], dtype=jnp.int32)
    return out_eq, out_lq.reshape(S), new_ptr


# speedup vs baseline: 39.2892x; 1.1080x over previous
import jax
import jax.numpy as jnp
from jax.experimental import pallas as pl
from jax.experimental.pallas import tpu as pltpu

_R = 512
_NBUF = 24
_K = 12


def _copy_body(emb, lab, eq, lq, out_eq, out_lq, vb, vlab, fsem, dsem, lsem):
    S, D = out_eq.shape
    B = emb.shape[0]
    nb = S // _R
    nb_emb = B // _R

    fills = [
        pltpu.make_async_copy(
            (emb if b < nb_emb else eq).at[pl.ds(b * _R, _R)],
            vb.at[b % _NBUF],
            fsem.at[b % _NBUF],
        )
        for b in range(nb)
    ]
    drains = [
        pltpu.make_async_copy(
            vb.at[b % _NBUF],
            out_eq.at[pl.ds(b * _R, _R)],
            dsem.at[b % _NBUF],
        )
        for b in range(nb)
    ]
    rl = lab.shape[0]
    ltail = lq.shape[0] - rl
    lfill1 = pltpu.make_async_copy(lab, vlab.at[pl.ds(0, rl)], lsem.at[0])
    lfill2 = pltpu.make_async_copy(
        lq.at[pl.ds(rl, ltail)], vlab.at[pl.ds(rl, ltail)], lsem.at[0]
    )
    ldrain = pltpu.make_async_copy(vlab, out_lq, lsem.at[1])

    lfill1.start()
    lfill2.start()
    for b in range(_NBUF):
        fills[b].start()
    lfill1.wait()
    lfill2.wait()
    ldrain.start()
    for b in range(nb):
        fills[b].wait()
        drains[b].start()
        j = b - _K
        if j >= 0 and j + _NBUF < nb:
            drains[j].wait()
            fills[j + _NBUF].start()
    waited = [j for j in range(nb) if j + _NBUF < nb and j <= nb - 1 - _K]
    first_unwaited = (waited[-1] + 1) if waited else 0
    for b in range(first_unwaited, nb):
        drains[b].wait()
    ldrain.wait()


def kernel(embeddings, labels, embed_queue, label_queue):
    B, D = embeddings.shape
    S = embed_queue.shape[0]
    lab2 = labels.reshape(B // 128, 128)
    lq2 = label_queue.reshape(S // 128, 128)
    out_eq, out_lq = pl.pallas_call(
        _copy_body,
        in_specs=[pl.BlockSpec(memory_space=pl.ANY)] * 4,
        out_specs=[pl.BlockSpec(memory_space=pl.ANY)] * 2,
        out_shape=[
            jax.ShapeDtypeStruct(embed_queue.shape, embed_queue.dtype),
            jax.ShapeDtypeStruct(lq2.shape, lq2.dtype),
        ],
        scratch_shapes=[
            pltpu.VMEM((_NBUF, _R, D), embed_queue.dtype),
            pltpu.VMEM((S // 128, 128), label_queue.dtype),
            pltpu.SemaphoreType.DMA((_NBUF,)),
            pltpu.SemaphoreType.DMA((_NBUF,)),
            pltpu.SemaphoreType.DMA((2,)),
        ],
    )(embeddings, lab2, embed_queue, lq2)
    new_ptr = jnp.array([B % S], dtype=jnp.int32)
    return out_eq, out_lq.reshape(S), new_ptr


# speedup vs baseline: 40.1536x; 1.0220x over previous
import jax
import jax.numpy as jnp
from jax.experimental import pallas as pl
from jax.experimental.pallas import tpu as pltpu

_R = 512
_NBUF = 32
_K = 16


def _copy_body(emb, lab, eq, lq, out_eq, out_lq, vb, vlab, fsem, dsem, lsem):
    S, D = out_eq.shape
    B = emb.shape[0]
    nb = S // _R
    nb_emb = B // _R

    fills = [
        pltpu.make_async_copy(
            (emb if b < nb_emb else eq).at[pl.ds(b * _R, _R)],
            vb.at[b % _NBUF],
            fsem.at[b % _NBUF],
        )
        for b in range(nb)
    ]
    drains = [
        pltpu.make_async_copy(
            vb.at[b % _NBUF],
            out_eq.at[pl.ds(b * _R, _R)],
            dsem.at[b % _NBUF],
        )
        for b in range(nb)
    ]
    rl = lab.shape[0]
    ltail = lq.shape[0] - rl
    lfill1 = pltpu.make_async_copy(lab, vlab.at[pl.ds(0, rl)], lsem.at[0])
    lfill2 = pltpu.make_async_copy(
        lq.at[pl.ds(rl, ltail)], vlab.at[pl.ds(rl, ltail)], lsem.at[0]
    )
    ldrain = pltpu.make_async_copy(vlab, out_lq, lsem.at[1])

    lfill1.start()
    lfill2.start()
    for b in range(_NBUF):
        fills[b].start()
    lfill1.wait()
    lfill2.wait()
    ldrain.start()
    for b in range(nb):
        fills[b].wait()
        drains[b].start()
        j = b - _K
        if j >= 0 and j + _NBUF < nb:
            drains[j].wait()
            fills[j + _NBUF].start()
    waited = [j for j in range(nb) if j + _NBUF < nb and j <= nb - 1 - _K]
    first_unwaited = (waited[-1] + 1) if waited else 0
    for b in range(first_unwaited, nb):
        drains[b].wait()
    ldrain.wait()


def kernel(embeddings, labels, embed_queue, label_queue):
    B, D = embeddings.shape
    S = embed_queue.shape[0]
    lab2 = labels.reshape(B // 128, 128)
    lq2 = label_queue.reshape(S // 128, 128)
    out_eq, out_lq = pl.pallas_call(
        _copy_body,
        in_specs=[pl.BlockSpec(memory_space=pl.ANY)] * 4,
        out_specs=[pl.BlockSpec(memory_space=pl.ANY)] * 2,
        out_shape=[
            jax.ShapeDtypeStruct(embed_queue.shape, embed_queue.dtype),
            jax.ShapeDtypeStruct(lq2.shape, lq2.dtype),
        ],
        scratch_shapes=[
            pltpu.VMEM((_NBUF, _R, D), embed_queue.dtype),
            pltpu.VMEM((S // 128, 128), label_queue.dtype),
            pltpu.SemaphoreType.DMA((_NBUF,)),
            pltpu.SemaphoreType.DMA((_NBUF,)),
            pltpu.SemaphoreType.DMA((2,)),
        ],
    )(embeddings, lab2, embed_queue, lq2)
    new_ptr = jnp.array([B % S], dtype=jnp.int32)
    return out_eq, out_lq.reshape(S), new_ptr


# speedup vs baseline: 41.3216x; 1.0291x over previous
import jax
import jax.numpy as jnp
from jax.experimental import pallas as pl
from jax.experimental.pallas import tpu as pltpu

_R = 512
_NBUF = 48
_K = 24


def _copy_body(emb, lab, eq, lq, out_eq, out_lq, vb, vlab, fsem, dsem, lsem):
    S, D = out_eq.shape
    B = emb.shape[0]
    nb = S // _R
    nb_emb = B // _R

    fills = [
        pltpu.make_async_copy(
            (emb if b < nb_emb else eq).at[pl.ds(b * _R, _R)],
            vb.at[b % _NBUF],
            fsem.at[b % _NBUF],
        )
        for b in range(nb)
    ]
    drains = [
        pltpu.make_async_copy(
            vb.at[b % _NBUF],
            out_eq.at[pl.ds(b * _R, _R)],
            dsem.at[b % _NBUF],
        )
        for b in range(nb)
    ]
    rl = lab.shape[0]
    ltail = lq.shape[0] - rl
    lfill1 = pltpu.make_async_copy(lab, vlab.at[pl.ds(0, rl)], lsem.at[0])
    lfill2 = pltpu.make_async_copy(
        lq.at[pl.ds(rl, ltail)], vlab.at[pl.ds(rl, ltail)], lsem.at[0]
    )
    ldrain = pltpu.make_async_copy(vlab, out_lq, lsem.at[1])

    lfill1.start()
    lfill2.start()
    for b in range(_NBUF):
        fills[b].start()
    lfill1.wait()
    lfill2.wait()
    ldrain.start()
    for b in range(nb):
        fills[b].wait()
        drains[b].start()
        j = b - _K
        if j >= 0 and j + _NBUF < nb:
            drains[j].wait()
            fills[j + _NBUF].start()
    waited = [j for j in range(nb) if j + _NBUF < nb and j <= nb - 1 - _K]
    first_unwaited = (waited[-1] + 1) if waited else 0
    for b in range(first_unwaited, nb):
        drains[b].wait()
    ldrain.wait()


def kernel(embeddings, labels, embed_queue, label_queue):
    B, D = embeddings.shape
    S = embed_queue.shape[0]
    lab2 = labels.reshape(B // 128, 128)
    lq2 = label_queue.reshape(S // 128, 128)
    out_eq, out_lq = pl.pallas_call(
        _copy_body,
        in_specs=[pl.BlockSpec(memory_space=pl.ANY)] * 4,
        out_specs=[pl.BlockSpec(memory_space=pl.ANY)] * 2,
        out_shape=[
            jax.ShapeDtypeStruct(embed_queue.shape, embed_queue.dtype),
            jax.ShapeDtypeStruct(lq2.shape, lq2.dtype),
        ],
        scratch_shapes=[
            pltpu.VMEM((_NBUF, _R, D), embed_queue.dtype),
            pltpu.VMEM((S // 128, 128), label_queue.dtype),
            pltpu.SemaphoreType.DMA((_NBUF,)),
            pltpu.SemaphoreType.DMA((_NBUF,)),
            pltpu.SemaphoreType.DMA((2,)),
        ],
    )(embeddings, lab2, embed_queue, lq2)
    new_ptr = jnp.array([B % S], dtype=jnp.int32)
    return out_eq, out_lq.reshape(S), new_ptr


# speedup vs baseline: 41.6647x; 1.0083x over previous
import jax
import jax.numpy as jnp
from jax.experimental import pallas as pl
from jax.experimental.pallas import tpu as pltpu

_R = 512
_NBUF = 64
_K = 32


def _copy_body(emb, lab, eq, lq, out_eq, out_lq, vb, vlab, fsem, dsem, lsem):
    S, D = out_eq.shape
    B = emb.shape[0]
    nb = S // _R
    nb_emb = B // _R

    fills = [
        pltpu.make_async_copy(
            (emb if b < nb_emb else eq).at[pl.ds(b * _R, _R)],
            vb.at[b % _NBUF],
            fsem.at[b % _NBUF],
        )
        for b in range(nb)
    ]
    drains = [
        pltpu.make_async_copy(
            vb.at[b % _NBUF],
            out_eq.at[pl.ds(b * _R, _R)],
            dsem.at[b % _NBUF],
        )
        for b in range(nb)
    ]
    rl = lab.shape[0]
    ltail = lq.shape[0] - rl
    lfill1 = pltpu.make_async_copy(lab, vlab.at[pl.ds(0, rl)], lsem.at[0])
    lfill2 = pltpu.make_async_copy(
        lq.at[pl.ds(rl, ltail)], vlab.at[pl.ds(rl, ltail)], lsem.at[0]
    )
    ldrain = pltpu.make_async_copy(vlab, out_lq, lsem.at[1])

    lfill1.start()
    lfill2.start()
    for b in range(_NBUF):
        fills[b].start()
    lfill1.wait()
    lfill2.wait()
    ldrain.start()
    for b in range(nb):
        fills[b].wait()
        drains[b].start()
        j = b - _K
        if j >= 0 and j + _NBUF < nb:
            drains[j].wait()
            fills[j + _NBUF].start()
    waited = [j for j in range(nb) if j + _NBUF < nb and j <= nb - 1 - _K]
    first_unwaited = (waited[-1] + 1) if waited else 0
    for b in range(first_unwaited, nb):
        drains[b].wait()
    ldrain.wait()


def kernel(embeddings, labels, embed_queue, label_queue):
    B, D = embeddings.shape
    S = embed_queue.shape[0]
    lab2 = labels.reshape(B // 128, 128)
    lq2 = label_queue.reshape(S // 128, 128)
    out_eq, out_lq = pl.pallas_call(
        _copy_body,
        in_specs=[pl.BlockSpec(memory_space=pl.ANY)] * 4,
        out_specs=[pl.BlockSpec(memory_space=pl.ANY)] * 2,
        out_shape=[
            jax.ShapeDtypeStruct(embed_queue.shape, embed_queue.dtype),
            jax.ShapeDtypeStruct(lq2.shape, lq2.dtype),
        ],
        scratch_shapes=[
            pltpu.VMEM((_NBUF, _R, D), embed_queue.dtype),
            pltpu.VMEM((S // 128, 128), label_queue.dtype),
            pltpu.SemaphoreType.DMA((_NBUF,)),
            pltpu.SemaphoreType.DMA((_NBUF,)),
            pltpu.SemaphoreType.DMA((2,)),
        ],
    )(embeddings, lab2, embed_queue, lq2)
    new_ptr = jnp.array([B % S], dtype=jnp.int32)
    return out_eq, out_lq.reshape(S), new_ptr
